# baseline (speedup 1.0000x reference)
import functools

import jax
import jax.numpy as jnp
from jax import lax
from jax.experimental import pallas as pl
from jax.experimental.pallas import tpu as pltpu
from jax.experimental.pallas import tpu_sc as plsc

D = 768
LANES = 16
ROW_V = D // LANES

NC = 2
NS = 16
NW = NC * NS
CHUNK = 32
NBUF = 3


def _make_emb_kernel(B: int, S: int):
    BS = B * S
    pos_w = S // NW
    halves = pos_w // CHUNK
    nt = B * halves
    per_w = BS // NW

    mesh = plsc.VectorSubcoreMesh(core_axis_name="c", subcore_axis_name="s")

    scratch = [pltpu.VMEM((nt, CHUNK), jnp.int32)]
    scratch += [pltpu.VMEM((CHUNK, D), jnp.float32) for _ in range(NBUF)]
    scratch += [pltpu.VMEM((pos_w, D), jnp.float32),
                pltpu.VMEM((per_w,), jnp.float32),
                pltpu.VMEM((per_w,), jnp.float32)]
    scratch += [pltpu.SemaphoreType.DMA for _ in range(2 * NBUF + 1)]

    @functools.partial(
        pl.kernel,
        mesh=mesh,
        out_type=[
            jax.ShapeDtypeStruct((BS, D), jnp.float32),
            jax.ShapeDtypeStruct((BS,), jnp.float32),
        ],
        scratch_types=scratch,
    )
    def emb_kernel(ids_hbm, mask_hbm, wte_hbm, wpe_hbm,
                   out_hbm, am_hbm, idx_v, *rest):
        bufs = rest[:NBUF]
        wpe_v, mask_v, am_v = rest[NBUF], rest[NBUF + 1], rest[NBUF + 2]
        sem_g = rest[NBUF + 3:NBUF + 3 + NBUF]
        sem_o = rest[NBUF + 3 + NBUF:NBUF + 3 + 2 * NBUF]
        sem_w = rest[NBUF + 3 + 2 * NBUF]

        wid = lax.axis_index("s") * NC + lax.axis_index("c")
        mbase = wid * per_w
        pbase = wid * pos_w

        pltpu.sync_copy(ids_hbm.at[wid], idx_v)

        gt = [None] * nt
        out_cp = [None] * NBUF
        for t in range(NBUF - 1):
            gt[t] = pltpu.async_copy(
                wte_hbm.at[idx_v.at[t]], bufs[t % NBUF], sem_g[t % NBUF])

        wpe_cp = pltpu.async_copy(
            wpe_hbm.at[pl.ds(pbase, pos_w)], wpe_v, sem_w)

        pltpu.sync_copy(mask_hbm.at[pl.ds(mbase, per_w)], mask_v)
        neg_inf = jnp.float32(jnp.finfo(jnp.float32).min)
        for i in range(per_w // LANES):
            m = mask_v[pl.ds(i * LANES, LANES)]
            am_v[pl.ds(i * LANES, LANES)] = (1.0 - m) * neg_inf
        pltpu.sync_copy(am_v, am_hbm.at[pl.ds(mbase, per_w)])
        wpe_cp.wait()

        def row_of(t):
            b, h = divmod(t, halves)
            return b * S + pbase + h * CHUNK

        for t in range(nt + NBUF - 1):
            if NBUF - 1 <= t < nt:
                p = t % NBUF
                if out_cp[p] is not None:
                    out_cp[p].wait()
                gt[t] = pltpu.async_copy(
                    wte_hbm.at[idx_v.at[t]], bufs[p], sem_g[p])
            u = t - (NBUF - 1)
            if 0 <= u < nt:
                p = u % NBUF
                gt[u].wait()
                h = u % halves
                buf = bufs[p]

                @plsc.parallel_loop(0, CHUNK, unroll=4)
                def add_row(r):
                    for j in range(ROW_V):
                        sl = pl.ds(j * LANES, LANES)
                        plsc.addupdate(buf.at[r, sl],
                                       wpe_v[h * CHUNK + r, sl])

                out_cp[p] = pltpu.async_copy(
                    buf, out_hbm.at[pl.ds(row_of(u), CHUNK)], sem_o[p])
        for p in range(NBUF):
            if out_cp[p] is not None:
                out_cp[p].wait()

    return emb_kernel


def kernel(input_ids, attention_mask, wte, wpe):
    input_shape = input_ids.shape
    S = input_shape[-1]
    ids2 = input_ids.reshape(-1, S)
    B = ids2.shape[0]
    BS = B * S

    pos_w = S // NW
    halves = pos_w // CHUNK
    ids_t = (ids2.reshape(B, NW, halves, CHUNK)
             .transpose(1, 0, 2, 3)
             .reshape(NW, B * halves, CHUNK)
             .astype(jnp.int32))
    mask_flat = attention_mask.reshape(BS).astype(jnp.float32)

    hidden, am = _make_emb_kernel(B, S)(ids_t, mask_flat, wte, wpe)
    hidden = hidden.reshape(B, S, D)
    am = am.reshape(B, 1, 1, S)
    return (hidden, am)


# speedup vs baseline: 1.0741x; 1.0741x over previous
import functools

import jax
import jax.numpy as jnp
from jax import lax
from jax.experimental import pallas as pl
from jax.experimental.pallas import tpu as pltpu
from jax.experimental.pallas import tpu_sc as plsc

D = 768
LANES = 16
ROW_V = D // LANES

NC = 2
NS = 16
NW = NC * NS
CHUNK = 32
NBUF = 3


def _make_emb_kernel(B: int, S: int):
    BS = B * S
    pos_w = S // NW
    halves = pos_w // CHUNK
    nt = B * halves
    per_w = BS // NW

    mesh = plsc.VectorSubcoreMesh(core_axis_name="c", subcore_axis_name="s")

    scratch = [pltpu.VMEM((nt, CHUNK), jnp.int32)]
    scratch += [pltpu.VMEM((CHUNK, D), jnp.float32) for _ in range(NBUF)]
    scratch += [pltpu.VMEM((pos_w, D), jnp.float32),
                pltpu.VMEM((per_w,), jnp.float32),
                pltpu.VMEM((per_w,), jnp.float32)]
    scratch += [pltpu.SemaphoreType.DMA for _ in range(2 * NBUF + 1)]

    @functools.partial(
        pl.kernel,
        mesh=mesh,
        out_type=[
            jax.ShapeDtypeStruct((BS, D), jnp.float32),
            jax.ShapeDtypeStruct((BS,), jnp.float32),
        ],
        scratch_types=scratch,
    )
    def emb_kernel(ids_hbm, mask_hbm, wte_hbm, wpe_hbm,
                   out_hbm, am_hbm, idx_v, *rest):
        bufs = rest[:NBUF]
        wpe_v, mask_v, am_v = rest[NBUF], rest[NBUF + 1], rest[NBUF + 2]
        sem_g = rest[NBUF + 3:NBUF + 3 + NBUF]
        sem_o = rest[NBUF + 3 + NBUF:NBUF + 3 + 2 * NBUF]
        sem_w = rest[NBUF + 3 + 2 * NBUF]

        wid = lax.axis_index("s") * NC + lax.axis_index("c")
        mbase = wid * per_w
        pbase = wid * pos_w

        pltpu.sync_copy(ids_hbm.at[wid], idx_v)

        gt = [None] * nt
        out_cp = [None] * NBUF
        for t in range(NBUF - 1):
            gt[t] = pltpu.async_copy(
                wte_hbm.at[idx_v.at[t]], bufs[t % NBUF], sem_g[t % NBUF])

        wpe_cp = pltpu.async_copy(
            wpe_hbm.at[pl.ds(pbase, pos_w)], wpe_v, sem_w)

        pltpu.sync_copy(mask_hbm.at[pl.ds(mbase, per_w)], mask_v)
        neg_inf = jnp.float32(jnp.finfo(jnp.float32).min)
        for i in range(per_w // LANES):
            m = mask_v[pl.ds(i * LANES, LANES)]
            am_v[pl.ds(i * LANES, LANES)] = (1.0 - m) * neg_inf
        pltpu.sync_copy(am_v, am_hbm.at[pl.ds(mbase, per_w)])
        wpe_cp.wait()

        def row_of(t):
            b, h = divmod(t, halves)
            return b * S + pbase + h * CHUNK

        for t in range(nt + NBUF - 1):
            if NBUF - 1 <= t < nt:
                p = t % NBUF
                if out_cp[p] is not None:
                    out_cp[p].wait()
                gt[t] = pltpu.async_copy(
                    wte_hbm.at[idx_v.at[t]], bufs[p], sem_g[p])
            u = t - (NBUF - 1)
            if 0 <= u < nt:
                p = u % NBUF
                gt[u].wait()
                h = u % halves
                buf = bufs[p]

                @plsc.parallel_loop(0, CHUNK, unroll=2)
                def add_row(r):
                    for j in range(ROW_V):
                        sl = pl.ds(j * LANES, LANES)
                        plsc.addupdate(buf.at[r, sl],
                                       wpe_v[h * CHUNK + r, sl])

                out_cp[p] = pltpu.async_copy(
                    buf, out_hbm.at[pl.ds(row_of(u), CHUNK)], sem_o[p])
        for p in range(NBUF):
            if out_cp[p] is not None:
                out_cp[p].wait()

    return emb_kernel


def kernel(input_ids, attention_mask, wte, wpe):
    input_shape = input_ids.shape
    S = input_shape[-1]
    ids2 = input_ids.reshape(-1, S)
    B = ids2.shape[0]
    BS = B * S

    pos_w = S // NW
    halves = pos_w // CHUNK
    ids_t = (ids2.reshape(B, NW, halves, CHUNK)
             .transpose(1, 0, 2, 3)
             .reshape(NW, B * halves, CHUNK)
             .astype(jnp.int32))
    mask_flat = attention_mask.reshape(BS).astype(jnp.float32)

    hidden, am = _make_emb_kernel(B, S)(ids_t, mask_flat, wte, wpe)
    hidden = hidden.reshape(B, S, D)
    am = am.reshape(B, 1, 1, S)
    return (hidden, am)


# speedup vs baseline: 1.1325x; 1.0544x over previous
import functools

import jax
import jax.numpy as jnp
from jax import lax
from jax.experimental import pallas as pl
from jax.experimental.pallas import tpu as pltpu
from jax.experimental.pallas import tpu_sc as plsc

D = 768
LANES = 16
ROW_V = D // LANES

NC = 2
NS = 16
NW = NC * NS
CHUNK = 32
NBUF = 3


def _make_emb_kernel(B: int, S: int):
    BS = B * S
    pos_w = S // NW
    halves = pos_w // CHUNK
    nt = B * halves
    per_w = BS // NW

    mesh = plsc.VectorSubcoreMesh(core_axis_name="c", subcore_axis_name="s")

    scratch = [pltpu.VMEM((nt, CHUNK), jnp.int32)]
    scratch += [pltpu.VMEM((CHUNK, D), jnp.float32) for _ in range(NBUF)]
    scratch += [pltpu.VMEM((pos_w, D), jnp.float32),
                pltpu.VMEM((per_w,), jnp.float32),
                pltpu.VMEM((per_w,), jnp.float32)]
    scratch += [pltpu.SemaphoreType.DMA for _ in range(2 * NBUF + 1)]

    @functools.partial(
        pl.kernel,
        mesh=mesh,
        out_type=[
            jax.ShapeDtypeStruct((BS, D), jnp.float32),
            jax.ShapeDtypeStruct((BS,), jnp.float32),
        ],
        scratch_types=scratch,
    )
    def emb_kernel(ids_hbm, mask_hbm, wte_hbm, wpe_hbm,
                   out_hbm, am_hbm, idx_v, *rest):
        bufs = rest[:NBUF]
        wpe_v, mask_v, am_v = rest[NBUF], rest[NBUF + 1], rest[NBUF + 2]
        sem_g = rest[NBUF + 3:NBUF + 3 + NBUF]
        sem_o = rest[NBUF + 3 + NBUF:NBUF + 3 + 2 * NBUF]
        sem_w = rest[NBUF + 3 + 2 * NBUF]

        wid = lax.axis_index("s") * NC + lax.axis_index("c")
        mbase = wid * per_w
        pbase = wid * pos_w

        pltpu.sync_copy(ids_hbm.at[wid], idx_v)

        gt = [None] * nt
        out_cp = [None] * NBUF
        for t in range(NBUF - 1):
            gt[t] = pltpu.async_copy(
                wte_hbm.at[idx_v.at[t]], bufs[t % NBUF], sem_g[t % NBUF])

        wpe_cp = pltpu.async_copy(
            wpe_hbm.at[pl.ds(pbase, pos_w)], wpe_v, sem_w)

        pltpu.sync_copy(mask_hbm.at[pl.ds(mbase, per_w)], mask_v)
        neg_inf = jnp.float32(jnp.finfo(jnp.float32).min)
        for i in range(per_w // LANES):
            m = mask_v[pl.ds(i * LANES, LANES)]
            am_v[pl.ds(i * LANES, LANES)] = (1.0 - m) * neg_inf
        pltpu.sync_copy(am_v, am_hbm.at[pl.ds(mbase, per_w)])
        wpe_cp.wait()

        def row_of(t):
            b, h = divmod(t, halves)
            return b * S + pbase + h * CHUNK

        for t in range(nt + NBUF - 1):
            if NBUF - 1 <= t < nt:
                p = t % NBUF
                if out_cp[p] is not None:
                    out_cp[p].wait()
                gt[t] = pltpu.async_copy(
                    wte_hbm.at[idx_v.at[t]], bufs[p], sem_g[p])
            u = t - (NBUF - 1)
            if 0 <= u < nt:
                p = u % NBUF
                gt[u].wait()
                h = u % halves
                buf = bufs[p]

                @plsc.parallel_loop(0, CHUNK, unroll=1)
                def add_row(r):
                    for j in range(ROW_V):
                        sl = pl.ds(j * LANES, LANES)
                        plsc.addupdate(buf.at[r, sl],
                                       wpe_v[h * CHUNK + r, sl])

                out_cp[p] = pltpu.async_copy(
                    buf, out_hbm.at[pl.ds(row_of(u), CHUNK)], sem_o[p])
        for p in range(NBUF):
            if out_cp[p] is not None:
                out_cp[p].wait()

    return emb_kernel


def kernel(input_ids, attention_mask, wte, wpe):
    input_shape = input_ids.shape
    S = input_shape[-1]
    ids2 = input_ids.reshape(-1, S)
    B = ids2.shape[0]
    BS = B * S

    pos_w = S // NW
    halves = pos_w // CHUNK
    ids_t = (ids2.reshape(B, NW, halves, CHUNK)
             .transpose(1, 0, 2, 3)
             .reshape(NW, B * halves, CHUNK)
             .astype(jnp.int32))
    mask_flat = attention_mask.reshape(BS).astype(jnp.float32)

    hidden, am = _make_emb_kernel(B, S)(ids_t, mask_flat, wte, wpe)
    hidden = hidden.reshape(B, S, D)
    am = am.reshape(B, 1, 1, S)
    return (hidden, am)


# speedup vs baseline: 1.1944x; 1.0547x over previous
import functools

import jax
import jax.numpy as jnp
from jax import lax
from jax.experimental import pallas as pl
from jax.experimental.pallas import tpu as pltpu
from jax.experimental.pallas import tpu_sc as plsc

D = 768
LANES = 16
ROW_V = D // LANES

NC = 2
NS = 16
NW = NC * NS
PCHUNK = 8
NBUF = 3


def _make_emb_kernel(B: int, S: int):
    BS = B * S
    pos_w = S // NW
    nt = pos_w // PCHUNK
    chunk = B * PCHUNK
    per_w = BS // NW

    mesh = plsc.VectorSubcoreMesh(core_axis_name="c", subcore_axis_name="s")

    scratch = [pltpu.VMEM((nt, chunk), jnp.int32)]
    scratch += [pltpu.VMEM((chunk, D), jnp.float32) for _ in range(NBUF)]
    scratch += [pltpu.VMEM((pos_w, D), jnp.float32),
                pltpu.VMEM((per_w,), jnp.float32),
                pltpu.VMEM((per_w,), jnp.float32)]
    scratch += [pltpu.SemaphoreType.DMA for _ in range(2 * NBUF + 1)]

    @functools.partial(
        pl.kernel,
        mesh=mesh,
        out_type=[
            jax.ShapeDtypeStruct((BS, D), jnp.float32),
            jax.ShapeDtypeStruct((BS,), jnp.float32),
        ],
        scratch_types=scratch,
    )
    def emb_kernel(ids_hbm, mask_hbm, wte_hbm, wpe_hbm,
                   out_hbm, am_hbm, idx_v, *rest):
        bufs = rest[:NBUF]
        wpe_v, mask_v, am_v = rest[NBUF], rest[NBUF + 1], rest[NBUF + 2]
        sem_g = rest[NBUF + 3:NBUF + 3 + NBUF]
        sem_o = rest[NBUF + 3 + NBUF:NBUF + 3 + 2 * NBUF]
        sem_w = rest[NBUF + 3 + 2 * NBUF]

        wid = lax.axis_index("s") * NC + lax.axis_index("c")
        mbase = wid * per_w
        pbase = wid * pos_w

        pltpu.sync_copy(ids_hbm.at[wid], idx_v)

        gt = [None] * nt
        out_cp = [[] for _ in range(NBUF)]
        for t in range(NBUF - 1):
            gt[t] = pltpu.async_copy(
                wte_hbm.at[idx_v.at[t]], bufs[t % NBUF], sem_g[t % NBUF])

        wpe_cp = pltpu.async_copy(
            wpe_hbm.at[pl.ds(pbase, pos_w)], wpe_v, sem_w)

        pltpu.sync_copy(mask_hbm.at[pl.ds(mbase, per_w)], mask_v)
        neg_inf = jnp.float32(jnp.finfo(jnp.float32).min)
        for i in range(per_w // LANES):
            m = mask_v[pl.ds(i * LANES, LANES)]
            am_v[pl.ds(i * LANES, LANES)] = (1.0 - m) * neg_inf
        pltpu.sync_copy(am_v, am_hbm.at[pl.ds(mbase, per_w)])
        wpe_cp.wait()

        for t in range(nt + NBUF - 1):
            if NBUF - 1 <= t < nt:
                p = t % NBUF
                for cp in out_cp[p]:
                    cp.wait()
                out_cp[p] = []
                gt[t] = pltpu.async_copy(
                    wte_hbm.at[idx_v.at[t]], bufs[p], sem_g[p])
            u = t - (NBUF - 1)
            if 0 <= u < nt:
                p = u % NBUF
                gt[u].wait()
                buf = bufs[p]

                @plsc.parallel_loop(0, PCHUNK, unroll=1)
                def add_pos(r):
                    for j in range(ROW_V):
                        sl = pl.ds(j * LANES, LANES)
                        w = wpe_v[u * PCHUNK + r, sl]
                        for b in range(B):
                            plsc.addupdate(buf.at[b * PCHUNK + r, sl], w)

                out_cp[p] = [
                    pltpu.async_copy(
                        buf.at[pl.ds(b * PCHUNK, PCHUNK)],
                        out_hbm.at[pl.ds(b * S + pbase + u * PCHUNK,
                                         PCHUNK)],
                        sem_o[p])
                    for b in range(B)
                ]
        for p in range(NBUF):
            for cp in out_cp[p]:
                cp.wait()

    return emb_kernel


def kernel(input_ids, attention_mask, wte, wpe):
    input_shape = input_ids.shape
    S = input_shape[-1]
    ids2 = input_ids.reshape(-1, S)
    B = ids2.shape[0]
    BS = B * S

    pos_w = S // NW
    nt = pos_w // PCHUNK
    ids_t = (ids2.reshape(B, NW, nt, PCHUNK)
             .transpose(1, 2, 0, 3)
             .reshape(NW, nt, B * PCHUNK)
             .astype(jnp.int32))
    mask_flat = attention_mask.reshape(BS).astype(jnp.float32)

    hidden, am = _make_emb_kernel(B, S)(ids_t, mask_flat, wte, wpe)
    hidden = hidden.reshape(B, S, D)
    am = am.reshape(B, 1, 1, S)
    return (hidden, am)
